# Initial kernel scaffold; baseline (speedup 1.0000x reference)
import jax
import jax.numpy as jnp
from jax.experimental import pallas as pl


def kernel(x_sparse, tables):
    raise NotImplementedError("write your pallas kernel here")


# speedup vs baseline: 1.1448x; 1.1448x over previous
import functools

import jax
import jax.numpy as jnp
from jax import lax
from jax.experimental import pallas as pl
from jax.experimental.pallas import tpu as pltpu
from jax.experimental.pallas import tpu_sc as plsc

F = 26
V = 100000
D = 32
B = 16384

NC, NS = 2, 16
NW = NC * NS
ROWS = B * F
RPW = ROWS // NW
GSZ = 128
GPW = RPW // GSZ
CH_G = 8
CH_ROWS = CH_G * GSZ
NCH = GPW // CH_G

@functools.cache
def _build():
    mesh = plsc.VectorSubcoreMesh(
        core_axis_name="c", subcore_axis_name="s", num_cores=NC, num_subcores=NS
    )
    return functools.partial(
        pl.kernel,
        out_type=jax.ShapeDtypeStruct((ROWS, D), jnp.float32),
        mesh=mesh,
        scratch_types=[
            pltpu.VMEM((GPW, GSZ), jnp.int32),
            pltpu.VMEM((CH_ROWS, D), jnp.float32),
            pltpu.SemaphoreType.DMA,
        ],
        compiler_params=pltpu.CompilerParams(use_tc_tiling_on_sc=False),
    )(_embed_gather)


def _embed_gather(x_hbm, tab_hbm, out_hbm, idx_v, rows_v, sem):
    wid = lax.axis_index("s") * NC + lax.axis_index("c")
    base_row = wid * RPW

    pltpu.sync_copy(x_hbm.at[pl.ds(wid * GPW, GPW)], idx_v)

    def add_off(j, carry):
        r = j // 8
        l = j - r * 8
        pos = r * GSZ + l * 16 + lax.iota(jnp.int32, 16)
        f = lax.rem(pos, F)
        sl = (r, pl.ds(l * 16, 16))
        idx_v[sl] = idx_v[sl] + f * V
        return carry

    lax.fori_loop(0, GPW * 8, add_off, 0)

    def chunk(c, carry):
        descs = []
        for g in range(CH_G):
            descs.append(
                pltpu.async_copy(
                    tab_hbm.at[idx_v.at[c * CH_G + g]],
                    rows_v.at[pl.ds(g * GSZ, GSZ)],
                    sem,
                )
            )
        for d in descs:
            d.wait()
        pltpu.sync_copy(rows_v, out_hbm.at[pl.ds(base_row + c * CH_ROWS, CH_ROWS)])
        return carry

    lax.fori_loop(0, NCH, chunk, 0)


def kernel(x_sparse, tables):
    x_flat = x_sparse.astype(jnp.int32).reshape(ROWS // GSZ, GSZ)
    tab_flat = tables.reshape(F * V, D)
    out = _build()(x_flat, tab_flat)
    return out.reshape(B, F, D)


# speedup vs baseline: 1.1513x; 1.0057x over previous
import functools

import jax
import jax.numpy as jnp
from jax import lax
from jax.experimental import pallas as pl
from jax.experimental.pallas import tpu as pltpu
from jax.experimental.pallas import tpu_sc as plsc

F = 26
V = 100000
D = 32
B = 16384

NC, NS = 2, 16
NW = NC * NS
ROWS = B * F
RPW = ROWS // NW
CH = 1024
NCH = RPW // CH


@functools.cache
def _build():
    mesh = plsc.VectorSubcoreMesh(
        core_axis_name="c", subcore_axis_name="s", num_cores=NC, num_subcores=NS
    )
    return functools.partial(
        pl.kernel,
        out_type=jax.ShapeDtypeStruct((ROWS, D), jnp.float32),
        mesh=mesh,
        scratch_types=[
            pltpu.VMEM((RPW,), jnp.int32),
            pltpu.VMEM((CH, D), jnp.float32),
            pltpu.VMEM((CH, D), jnp.float32),
            pltpu.SemaphoreType.DMA,
            pltpu.SemaphoreType.DMA,
            pltpu.SemaphoreType.DMA,
            pltpu.SemaphoreType.DMA,
        ],
        compiler_params=pltpu.CompilerParams(use_tc_tiling_on_sc=False),
    )(_embed_gather)


def _embed_gather(x_hbm, tab_hbm, out_hbm, idx_v, rows0, rows1, g0, g1, w0, w1):
    wid = lax.axis_index("s") * NC + lax.axis_index("c")
    base_row = wid * RPW

    pltpu.sync_copy(x_hbm.at[pl.ds(base_row, RPW)], idx_v)

    def add_off(j, carry):
        pos = j * 16 + lax.iota(jnp.int32, 16)
        f = lax.rem(pos, F)
        sl = pl.ds(j * 16, 16)
        idx_v[sl] = idx_v[sl] + f * V
        return carry

    lax.fori_loop(0, RPW // 16, add_off, 0)

    rows = (rows0, rows1)
    gsem = (g0, g1)
    wsem = (w0, w1)

    def fire_gather(c, nb):
        pltpu.async_copy(
            tab_hbm.at[idx_v.at[pl.ds(c * CH, CH)]], rows[nb], gsem[nb]
        )

    def wait_gather(c, nb):
        pltpu.make_async_copy(
            tab_hbm.at[idx_v.at[pl.ds(c * CH, CH)]], rows[nb], gsem[nb]
        ).wait()

    def fire_write(c, nb):
        pltpu.async_copy(
            rows[nb], out_hbm.at[pl.ds(base_row + c * CH, CH)], wsem[nb]
        )

    def wait_write(c, nb):
        pltpu.make_async_copy(
            rows[nb], out_hbm.at[pl.ds(base_row + c * CH, CH)], wsem[nb]
        ).wait()

    fire_gather(0, 0)

    def step(c, nb, first=False):
        if first:
            @pl.when(c >= 1)
            def _():
                wait_write(c - 1, 1 - nb)
        else:
            wait_write(c - 1, 1 - nb)
        fire_gather(c + 1, 1 - nb)
        wait_gather(c, nb)
        fire_write(c, nb)

    def pair(k, carry):
        c = k * 2
        step(c, 0, first=True)
        step(c + 1, 1)
        return carry

    lax.fori_loop(0, (NCH - 1) // 2, pair, 0)
    last = NCH - 1
    wait_write(last - 1, 1)
    wait_gather(last, 0)
    fire_write(last, 0)
    wait_write(last, 0)


def kernel(x_sparse, tables):
    x_flat = x_sparse.astype(jnp.int32).reshape(ROWS)
    tab_flat = tables.reshape(F * V, D)
    out = _build()(x_flat, tab_flat)
    return out.reshape(B, F, D)


# speedup vs baseline: 1.1893x; 1.0330x over previous
import functools

import jax
import jax.numpy as jnp
from jax import lax
from jax.experimental import pallas as pl
from jax.experimental.pallas import tpu as pltpu
from jax.experimental.pallas import tpu_sc as plsc

F = 26
V = 100000
D = 32
B = 16384

NC, NS = 2, 16
NW = NC * NS
BPW = B // NW


@functools.cache
def _build():
    mesh = plsc.VectorSubcoreMesh(
        core_axis_name="c", subcore_axis_name="s", num_cores=NC, num_subcores=NS
    )
    return functools.partial(
        pl.kernel,
        out_type=jax.ShapeDtypeStruct((F, B, D), jnp.float32),
        mesh=mesh,
        scratch_types=[
            pltpu.VMEM((F, BPW), jnp.int32),
            pltpu.VMEM((BPW, D), jnp.float32),
            pltpu.VMEM((BPW, D), jnp.float32),
            pltpu.SemaphoreType.DMA,
            pltpu.SemaphoreType.DMA,
            pltpu.SemaphoreType.DMA,
            pltpu.SemaphoreType.DMA,
        ],
        compiler_params=pltpu.CompilerParams(use_tc_tiling_on_sc=False),
    )(_embed_gather)


def _embed_gather(xt_hbm, tab_hbm, out_hbm, idx_v, rows0, rows1, g0, g1, w0, w1):
    wid = lax.axis_index("s") * NC + lax.axis_index("c")
    b0 = wid * BPW

    pltpu.sync_copy(xt_hbm.at[:, pl.ds(b0, BPW)], idx_v)

    def add_off(j, carry):
        f = j // (BPW // 16)
        l = j - f * (BPW // 16)
        sl = (f, pl.ds(l * 16, 16))
        idx_v[sl] = idx_v[sl] + f * V
        return carry

    lax.fori_loop(0, F * (BPW // 16), add_off, 0)

    rows = (rows0, rows1)
    gsem = (g0, g1)
    wsem = (w0, w1)

    def fire_gather(f, nb):
        pltpu.async_copy(tab_hbm.at[idx_v.at[f]], rows[nb], gsem[nb])

    def wait_gather(f, nb):
        pltpu.make_async_copy(tab_hbm.at[idx_v.at[f]], rows[nb], gsem[nb]).wait()

    def fire_write(f, nb):
        pltpu.async_copy(rows[nb], out_hbm.at[f, pl.ds(b0, BPW), :], wsem[nb])

    def wait_write(f, nb):
        pltpu.make_async_copy(
            rows[nb], out_hbm.at[f, pl.ds(b0, BPW), :], wsem[nb]
        ).wait()

    fire_gather(0, 0)

    def step(c, nb, first=False):
        if first:
            @pl.when(c >= 1)
            def _():
                wait_write(c - 1, 1 - nb)
        else:
            wait_write(c - 1, 1 - nb)
        fire_gather(c + 1, 1 - nb)
        wait_gather(c, nb)
        fire_write(c, nb)

    def pair(k, carry):
        c = k * 2
        step(c, 0, first=True)
        step(c + 1, 1)
        return carry

    lax.fori_loop(0, (F - 2) // 2, pair, 0)
    step(F - 2, 0)
    wait_gather(F - 1, 1)
    fire_write(F - 1, 1)
    wait_write(F - 2, 0)
    wait_write(F - 1, 1)


def kernel(x_sparse, tables):
    xt = jnp.transpose(x_sparse.astype(jnp.int32))
    tab_flat = tables.reshape(F * V, D)
    out_fmajor = _build()(xt, tab_flat)
    return jnp.transpose(out_fmajor, (1, 0, 2))
